# initial kernel scaffold (unmeasured)
import jax
import jax.numpy as jnp
from jax import lax
from jax.experimental import pallas as pl
from jax.experimental.pallas import tpu as pltpu


def kernel(
    x,
):
    def body(*refs):
        pass

    out_shape = jax.ShapeDtypeStruct(..., jnp.float32)
    return pl.pallas_call(body, out_shape=out_shape)(...)


# baseline (device time: 259978 ns/iter reference)
import jax
import jax.numpy as jnp
from jax import lax
from jax.experimental import pallas as pl
from jax.experimental.pallas import tpu as pltpu

N_DEV = 16
M = 4096
N = 1024
CH = M // N_DEV


def kernel(x):
    def body(
        x_ref,
        out_ref,
        rs_send,
        rs_recv,
        ag_buf,
        rs_send_sems,
        rs_recv_sems,
        ag_send_sems,
        ag_recv_sems,
    ):
        my = lax.axis_index("i")
        right = lax.rem(my + 1, N_DEV)

        def x_chunk_bf16(c):
            return x_ref[pl.ds(c * CH, CH), :].astype(jnp.bfloat16)

        rs_send[0, :, :] = x_chunk_bf16(my)
        for h in range(N_DEV - 1):
            rdma = pltpu.make_async_remote_copy(
                src_ref=rs_send.at[h],
                dst_ref=rs_recv.at[h],
                send_sem=rs_send_sems.at[h],
                recv_sem=rs_recv_sems.at[h],
                device_id=(right,),
                device_id_type=pl.DeviceIdType.MESH,
            )
            rdma.start()
            rdma.wait()
            c = lax.rem(my - (h + 1) + 2 * N_DEV, N_DEV)
            if h < N_DEV - 2:
                rs_send[h + 1, :, :] = rs_recv[h] + x_chunk_bf16(c)
            else:
                ag_buf[N_DEV - 1, :, :] = rs_recv[h] + x_chunk_bf16(c)

        own = lax.rem(my + 1, N_DEV)
        out_ref[pl.ds(own * CH, CH), :] = ag_buf[N_DEV - 1].astype(jnp.float32)

        for h in range(N_DEV - 1):
            src = ag_buf.at[N_DEV - 1] if h == 0 else ag_buf.at[h - 1]
            rdma = pltpu.make_async_remote_copy(
                src_ref=src,
                dst_ref=ag_buf.at[h],
                send_sem=ag_send_sems.at[h],
                recv_sem=ag_recv_sems.at[h],
                device_id=(right,),
                device_id_type=pl.DeviceIdType.MESH,
            )
            rdma.start()
            rdma.wait()
            c = lax.rem(my - h + 2 * N_DEV, N_DEV)
            out_ref[pl.ds(c * CH, CH), :] = ag_buf[h].astype(jnp.float32)

    return pl.pallas_call(
        body,
        out_shape=jax.ShapeDtypeStruct((M, N), jnp.float32),
        in_specs=[pl.BlockSpec(memory_space=pltpu.VMEM)],
        out_specs=pl.BlockSpec(memory_space=pltpu.VMEM),
        scratch_shapes=[
            pltpu.VMEM((N_DEV - 1, CH, N), jnp.bfloat16),
            pltpu.VMEM((N_DEV - 1, CH, N), jnp.bfloat16),
            pltpu.VMEM((N_DEV, CH, N), jnp.bfloat16),
            pltpu.SemaphoreType.DMA((N_DEV - 1,)),
            pltpu.SemaphoreType.DMA((N_DEV - 1,)),
            pltpu.SemaphoreType.DMA((N_DEV - 1,)),
            pltpu.SemaphoreType.DMA((N_DEV - 1,)),
        ],
        compiler_params=pltpu.CompilerParams(
            vmem_limit_bytes=100 * 1024 * 1024,
        ),
    )(x)


# device time: 212388 ns/iter; 1.2241x vs baseline; 1.2241x over previous
import jax
import jax.numpy as jnp
from jax import lax
from jax.experimental import pallas as pl
from jax.experimental.pallas import tpu as pltpu

N_DEV = 16
M = 4096
N = 1024
CH = M // N_DEV
HN = N // 2


def kernel(x):
    def body(
        x_ref,
        out_ref,
        rs_send_f,
        rs_recv_f,
        rs_send_b,
        rs_recv_b,
        ag_f,
        ag_b,
        rs_ssem_f,
        rs_rsem_f,
        rs_ssem_b,
        rs_rsem_b,
        ag_ssem_f,
        ag_rsem_f,
        ag_ssem_b,
        ag_rsem_b,
    ):
        my = lax.axis_index("i")
        right = lax.rem(my + 1, N_DEV)
        left = lax.rem(my + N_DEV - 1, N_DEV)

        def xf(c):
            return x_ref[pl.ds(c * CH, CH), 0:HN].astype(jnp.bfloat16)

        def xb(c):
            return x_ref[pl.ds(c * CH, CH), HN:N].astype(jnp.bfloat16)

        def rs_rdma(h):
            f = pltpu.make_async_remote_copy(
                src_ref=rs_send_f.at[h],
                dst_ref=rs_recv_f.at[h],
                send_sem=rs_ssem_f.at[h],
                recv_sem=rs_rsem_f.at[h],
                device_id=(right,),
                device_id_type=pl.DeviceIdType.MESH,
            )
            b = pltpu.make_async_remote_copy(
                src_ref=rs_send_b.at[h],
                dst_ref=rs_recv_b.at[h],
                send_sem=rs_ssem_b.at[h],
                recv_sem=rs_rsem_b.at[h],
                device_id=(left,),
                device_id_type=pl.DeviceIdType.MESH,
            )
            return f, b

        def ag_rdma(h):
            f = pltpu.make_async_remote_copy(
                src_ref=ag_f.at[N_DEV - 1] if h == 0 else ag_f.at[h - 1],
                dst_ref=ag_f.at[h],
                send_sem=ag_ssem_f.at[h],
                recv_sem=ag_rsem_f.at[h],
                device_id=(right,),
                device_id_type=pl.DeviceIdType.MESH,
            )
            b = pltpu.make_async_remote_copy(
                src_ref=ag_b.at[N_DEV - 1] if h == 0 else ag_b.at[h - 1],
                dst_ref=ag_b.at[h],
                send_sem=ag_ssem_b.at[h],
                recv_sem=ag_rsem_b.at[h],
                device_id=(left,),
                device_id_type=pl.DeviceIdType.MESH,
            )
            return f, b

        rs_send_f[0, :, :] = xf(my)
        rs_send_b[0, :, :] = xb(my)
        for h in range(N_DEV - 1):
            f, b = rs_rdma(h)
            f.start()
            b.start()
            cf = lax.rem(my - (h + 1) + 2 * N_DEV, N_DEV)
            cb = lax.rem(my + h + 1, N_DEV)
            f.wait_recv()
            if h < N_DEV - 2:
                rs_send_f[h + 1, :, :] = rs_recv_f[h] + xf(cf)
            else:
                ag_f[N_DEV - 1, :, :] = rs_recv_f[h] + xf(cf)
            b.wait_recv()
            if h < N_DEV - 2:
                rs_send_b[h + 1, :, :] = rs_recv_b[h] + xb(cb)
            else:
                ag_b[N_DEV - 1, :, :] = rs_recv_b[h] + xb(cb)

        own_f = lax.rem(my + 1, N_DEV)
        own_b = lax.rem(my + N_DEV - 1, N_DEV)
        out_ref[pl.ds(own_f * CH, CH), 0:HN] = ag_f[N_DEV - 1].astype(jnp.float32)
        out_ref[pl.ds(own_b * CH, CH), HN:N] = ag_b[N_DEV - 1].astype(jnp.float32)

        for h in range(N_DEV - 1):
            f, b = ag_rdma(h)
            f.start()
            b.start()
            cf = lax.rem(my - h + 2 * N_DEV, N_DEV)
            cb = lax.rem(my + h, N_DEV)
            f.wait_recv()
            out_ref[pl.ds(cf * CH, CH), 0:HN] = ag_f[h].astype(jnp.float32)
            b.wait_recv()
            out_ref[pl.ds(cb * CH, CH), HN:N] = ag_b[h].astype(jnp.float32)

        for h in range(N_DEV - 1):
            f, b = rs_rdma(h)
            f.wait_send()
            b.wait_send()
            f, b = ag_rdma(h)
            f.wait_send()
            b.wait_send()

    half = (N_DEV - 1, CH, HN)
    return pl.pallas_call(
        body,
        out_shape=jax.ShapeDtypeStruct((M, N), jnp.float32),
        in_specs=[pl.BlockSpec(memory_space=pltpu.VMEM)],
        out_specs=pl.BlockSpec(memory_space=pltpu.VMEM),
        scratch_shapes=[
            pltpu.VMEM(half, jnp.bfloat16),
            pltpu.VMEM(half, jnp.bfloat16),
            pltpu.VMEM(half, jnp.bfloat16),
            pltpu.VMEM(half, jnp.bfloat16),
            pltpu.VMEM((N_DEV, CH, HN), jnp.bfloat16),
            pltpu.VMEM((N_DEV, CH, HN), jnp.bfloat16),
            pltpu.SemaphoreType.DMA((N_DEV - 1,)),
            pltpu.SemaphoreType.DMA((N_DEV - 1,)),
            pltpu.SemaphoreType.DMA((N_DEV - 1,)),
            pltpu.SemaphoreType.DMA((N_DEV - 1,)),
            pltpu.SemaphoreType.DMA((N_DEV - 1,)),
            pltpu.SemaphoreType.DMA((N_DEV - 1,)),
            pltpu.SemaphoreType.DMA((N_DEV - 1,)),
            pltpu.SemaphoreType.DMA((N_DEV - 1,)),
        ],
        compiler_params=pltpu.CompilerParams(
            vmem_limit_bytes=100 * 1024 * 1024,
        ),
    )(x)


# device time: 172526 ns/iter; 1.5069x vs baseline; 1.2310x over previous
import jax
import jax.numpy as jnp
from jax import lax
from jax.experimental import pallas as pl
from jax.experimental.pallas import tpu as pltpu

N_DEV = 16
M = 4096
N = 1024
CH = M // N_DEV
HN = N // 2

_POS = [0, 1, 8, 9, 15, 2, 7, 10, 14, 3, 6, 11, 13, 4, 5, 12]
_SUCC = [1, 5, 3, 7, 0, 9, 2, 11, 4, 13, 6, 15, 8, 14, 10, 12]
_PRED = [4, 0, 6, 2, 8, 1, 10, 3, 12, 5, 14, 7, 15, 9, 13, 11]


def kernel(x):
    def body(
        x_ref,
        out_ref,
        xb16,
        rs_send_f,
        rs_recv_f,
        rs_send_b,
        rs_recv_b,
        ag_f,
        ag_b,
        rs_ssem_f,
        rs_rsem_f,
        rs_ssem_b,
        rs_rsem_b,
        ag_ssem_f,
        ag_rsem_f,
        ag_ssem_b,
        ag_rsem_b,
    ):
        my = lax.axis_index("i")

        def lut(table):
            v = jnp.int32(table[0])
            for k in range(1, N_DEV):
                v = jnp.where(my == k, jnp.int32(table[k]), v)
            return v

        r = lut(_POS)
        right = lut(_SUCC)
        left = lut(_PRED)

        xb16[:, :] = x_ref[:, :].astype(jnp.bfloat16)

        def xf(c):
            return xb16[pl.ds(c * CH, CH), 0:HN]

        def xb(c):
            return xb16[pl.ds(c * CH, CH), HN:N]

        def rs_rdma(h):
            f = pltpu.make_async_remote_copy(
                src_ref=rs_send_f.at[h],
                dst_ref=rs_recv_f.at[h],
                send_sem=rs_ssem_f.at[h],
                recv_sem=rs_rsem_f.at[h],
                device_id=(right,),
                device_id_type=pl.DeviceIdType.MESH,
            )
            b = pltpu.make_async_remote_copy(
                src_ref=rs_send_b.at[h],
                dst_ref=rs_recv_b.at[h],
                send_sem=rs_ssem_b.at[h],
                recv_sem=rs_rsem_b.at[h],
                device_id=(left,),
                device_id_type=pl.DeviceIdType.MESH,
            )
            return f, b

        def ag_rdma(h):
            f = pltpu.make_async_remote_copy(
                src_ref=ag_f.at[N_DEV - 1] if h == 0 else ag_f.at[h - 1],
                dst_ref=ag_f.at[h],
                send_sem=ag_ssem_f.at[h],
                recv_sem=ag_rsem_f.at[h],
                device_id=(right,),
                device_id_type=pl.DeviceIdType.MESH,
            )
            b = pltpu.make_async_remote_copy(
                src_ref=ag_b.at[N_DEV - 1] if h == 0 else ag_b.at[h - 1],
                dst_ref=ag_b.at[h],
                send_sem=ag_ssem_b.at[h],
                recv_sem=ag_rsem_b.at[h],
                device_id=(left,),
                device_id_type=pl.DeviceIdType.MESH,
            )
            return f, b

        rs_send_f[0, :, :] = xf(r)
        rs_send_b[0, :, :] = xb(r)
        for h in range(N_DEV - 1):
            f, b = rs_rdma(h)
            f.start()
            b.start()
            cf = lax.rem(r - (h + 1) + 2 * N_DEV, N_DEV)
            cb = lax.rem(r + h + 1, N_DEV)
            f.wait_recv()
            if h < N_DEV - 2:
                rs_send_f[h + 1, :, :] = rs_recv_f[h] + xf(cf)
            else:
                ag_f[N_DEV - 1, :, :] = rs_recv_f[h] + xf(cf)
            b.wait_recv()
            if h < N_DEV - 2:
                rs_send_b[h + 1, :, :] = rs_recv_b[h] + xb(cb)
            else:
                ag_b[N_DEV - 1, :, :] = rs_recv_b[h] + xb(cb)

        own_f = lax.rem(r + 1, N_DEV)
        own_b = lax.rem(r + N_DEV - 1, N_DEV)
        out_ref[pl.ds(own_f * CH, CH), 0:HN] = ag_f[N_DEV - 1].astype(jnp.float32)
        out_ref[pl.ds(own_b * CH, CH), HN:N] = ag_b[N_DEV - 1].astype(jnp.float32)

        for h in range(N_DEV - 1):
            f, b = ag_rdma(h)
            f.start()
            b.start()
            cf = lax.rem(r - h + 2 * N_DEV, N_DEV)
            cb = lax.rem(r + h, N_DEV)
            f.wait_recv()
            out_ref[pl.ds(cf * CH, CH), 0:HN] = ag_f[h].astype(jnp.float32)
            b.wait_recv()
            out_ref[pl.ds(cb * CH, CH), HN:N] = ag_b[h].astype(jnp.float32)

        for h in range(N_DEV - 1):
            f, b = rs_rdma(h)
            f.wait_send()
            b.wait_send()
            f, b = ag_rdma(h)
            f.wait_send()
            b.wait_send()

    half = (N_DEV - 1, CH, HN)
    return pl.pallas_call(
        body,
        out_shape=jax.ShapeDtypeStruct((M, N), jnp.float32),
        in_specs=[pl.BlockSpec(memory_space=pltpu.VMEM)],
        out_specs=pl.BlockSpec(memory_space=pltpu.VMEM),
        scratch_shapes=[
            pltpu.VMEM((M, N), jnp.bfloat16),
            pltpu.VMEM(half, jnp.bfloat16),
            pltpu.VMEM(half, jnp.bfloat16),
            pltpu.VMEM(half, jnp.bfloat16),
            pltpu.VMEM(half, jnp.bfloat16),
            pltpu.VMEM((N_DEV, CH, HN), jnp.bfloat16),
            pltpu.VMEM((N_DEV, CH, HN), jnp.bfloat16),
            pltpu.SemaphoreType.DMA((N_DEV - 1,)),
            pltpu.SemaphoreType.DMA((N_DEV - 1,)),
            pltpu.SemaphoreType.DMA((N_DEV - 1,)),
            pltpu.SemaphoreType.DMA((N_DEV - 1,)),
            pltpu.SemaphoreType.DMA((N_DEV - 1,)),
            pltpu.SemaphoreType.DMA((N_DEV - 1,)),
            pltpu.SemaphoreType.DMA((N_DEV - 1,)),
            pltpu.SemaphoreType.DMA((N_DEV - 1,)),
        ],
        compiler_params=pltpu.CompilerParams(
            vmem_limit_bytes=100 * 1024 * 1024,
        ),
    )(x)


# device time: 131335 ns/iter; 1.9795x vs baseline; 1.3136x over previous
import jax
import jax.numpy as jnp
from jax import lax
from jax.experimental import pallas as pl
from jax.experimental.pallas import tpu as pltpu

N_DEV = 16
M = 4096
N = 1024
CH = M // N_DEV
HN = N // 2
SUB = 2
SR = CH // SUB
NH = N_DEV - 1

_POS = [0, 1, 8, 9, 15, 2, 7, 10, 14, 3, 6, 11, 13, 4, 5, 12]
_SUCC = [1, 5, 3, 7, 0, 9, 2, 11, 4, 13, 6, 15, 8, 14, 10, 12]
_PRED = [4, 0, 6, 2, 8, 1, 10, 3, 12, 5, 14, 7, 15, 9, 13, 11]


def kernel(x):
    def body(
        x_ref,
        out_ref,
        xb16,
        rs_send_f,
        rs_recv_f,
        rs_send_b,
        rs_recv_b,
        ag_f,
        ag_b,
        rs_ssem_f,
        rs_rsem_f,
        rs_ssem_b,
        rs_rsem_b,
        ag_ssem_f,
        ag_rsem_f,
        ag_ssem_b,
        ag_rsem_b,
    ):
        my = lax.axis_index("i")

        def lut(table):
            v = jnp.int32(table[0])
            for k in range(1, N_DEV):
                v = jnp.where(my == k, jnp.int32(table[k]), v)
            return v

        r = lut(_POS)
        right = lut(_SUCC)
        left = lut(_PRED)

        xb16[:, :] = x_ref[:, :].astype(jnp.bfloat16)

        def xsub(c, s, fwd):
            cols = slice(0, HN) if fwd else slice(HN, N)
            return xb16[pl.ds(c * CH + s * SR, SR), cols]

        def rs_rdma(h, s, fwd):
            if fwd:
                return pltpu.make_async_remote_copy(
                    src_ref=rs_send_f.at[h, s],
                    dst_ref=rs_recv_f.at[h, s],
                    send_sem=rs_ssem_f.at[h * SUB + s],
                    recv_sem=rs_rsem_f.at[h * SUB + s],
                    device_id=(right,),
                    device_id_type=pl.DeviceIdType.MESH,
                )
            return pltpu.make_async_remote_copy(
                src_ref=rs_send_b.at[h, s],
                dst_ref=rs_recv_b.at[h, s],
                send_sem=rs_ssem_b.at[h * SUB + s],
                recv_sem=rs_rsem_b.at[h * SUB + s],
                device_id=(left,),
                device_id_type=pl.DeviceIdType.MESH,
            )

        def ag_rdma(h, s, fwd):
            if fwd:
                return pltpu.make_async_remote_copy(
                    src_ref=ag_f.at[NH if h == 0 else h - 1, s],
                    dst_ref=ag_f.at[h, s],
                    send_sem=ag_ssem_f.at[h * SUB + s],
                    recv_sem=ag_rsem_f.at[h * SUB + s],
                    device_id=(right,),
                    device_id_type=pl.DeviceIdType.MESH,
                )
            return pltpu.make_async_remote_copy(
                src_ref=ag_b.at[NH if h == 0 else h - 1, s],
                dst_ref=ag_b.at[h, s],
                send_sem=ag_ssem_b.at[h * SUB + s],
                recv_sem=ag_rsem_b.at[h * SUB + s],
                device_id=(left,),
                device_id_type=pl.DeviceIdType.MESH,
            )

        for s in range(SUB):
            rs_send_f[0, s, :, :] = xsub(r, s, True)
            rs_rdma(0, s, True).start()
            rs_send_b[0, s, :, :] = xsub(r, s, False)
            rs_rdma(0, s, False).start()

        for h in range(NH):
            cf = lax.rem(r - (h + 1) + 2 * N_DEV, N_DEV)
            cb = lax.rem(r + h + 1, N_DEV)
            for s in range(SUB):
                rs_rdma(h, s, True).wait_recv()
                if h < NH - 1:
                    rs_send_f[h + 1, s, :, :] = rs_recv_f[h, s] + xsub(cf, s, True)
                    rs_rdma(h + 1, s, True).start()
                else:
                    ag_f[NH, s, :, :] = rs_recv_f[h, s] + xsub(cf, s, True)
                    ag_rdma(0, s, True).start()
                rs_rdma(h, s, False).wait_recv()
                if h < NH - 1:
                    rs_send_b[h + 1, s, :, :] = rs_recv_b[h, s] + xsub(cb, s, False)
                    rs_rdma(h + 1, s, False).start()
                else:
                    ag_b[NH, s, :, :] = rs_recv_b[h, s] + xsub(cb, s, False)
                    ag_rdma(0, s, False).start()

        own_f = lax.rem(r + 1, N_DEV)
        own_b = lax.rem(r + N_DEV - 1, N_DEV)
        for s in range(SUB):
            out_ref[pl.ds(own_f * CH + s * SR, SR), 0:HN] = ag_f[NH, s].astype(
                jnp.float32
            )
            out_ref[pl.ds(own_b * CH + s * SR, SR), HN:N] = ag_b[NH, s].astype(
                jnp.float32
            )

        for h in range(NH):
            cf = lax.rem(r - h + 2 * N_DEV, N_DEV)
            cb = lax.rem(r + h, N_DEV)
            for s in range(SUB):
                ag_rdma(h, s, True).wait_recv()
                if h < NH - 1:
                    ag_rdma(h + 1, s, True).start()
                out_ref[pl.ds(cf * CH + s * SR, SR), 0:HN] = ag_f[h, s].astype(
                    jnp.float32
                )
                ag_rdma(h, s, False).wait_recv()
                if h < NH - 1:
                    ag_rdma(h + 1, s, False).start()
                out_ref[pl.ds(cb * CH + s * SR, SR), HN:N] = ag_b[h, s].astype(
                    jnp.float32
                )

        for h in range(NH):
            for s in range(SUB):
                rs_rdma(h, s, True).wait_send()
                rs_rdma(h, s, False).wait_send()
                ag_rdma(h, s, True).wait_send()
                ag_rdma(h, s, False).wait_send()

    rs_shape = (NH, SUB, SR, HN)
    ag_shape = (N_DEV, SUB, SR, HN)
    nsem = NH * SUB
    return pl.pallas_call(
        body,
        out_shape=jax.ShapeDtypeStruct((M, N), jnp.float32),
        in_specs=[pl.BlockSpec(memory_space=pltpu.VMEM)],
        out_specs=pl.BlockSpec(memory_space=pltpu.VMEM),
        scratch_shapes=[
            pltpu.VMEM((M, N), jnp.bfloat16),
            pltpu.VMEM(rs_shape, jnp.bfloat16),
            pltpu.VMEM(rs_shape, jnp.bfloat16),
            pltpu.VMEM(rs_shape, jnp.bfloat16),
            pltpu.VMEM(rs_shape, jnp.bfloat16),
            pltpu.VMEM(ag_shape, jnp.bfloat16),
            pltpu.VMEM(ag_shape, jnp.bfloat16),
            pltpu.SemaphoreType.DMA((nsem,)),
            pltpu.SemaphoreType.DMA((nsem,)),
            pltpu.SemaphoreType.DMA((nsem,)),
            pltpu.SemaphoreType.DMA((nsem,)),
            pltpu.SemaphoreType.DMA((nsem,)),
            pltpu.SemaphoreType.DMA((nsem,)),
            pltpu.SemaphoreType.DMA((nsem,)),
            pltpu.SemaphoreType.DMA((nsem,)),
        ],
        compiler_params=pltpu.CompilerParams(
            vmem_limit_bytes=100 * 1024 * 1024,
        ),
    )(x)


# device time: 130599 ns/iter; 1.9907x vs baseline; 1.0056x over previous
import jax
import jax.numpy as jnp
from jax import lax
from jax.experimental import pallas as pl
from jax.experimental.pallas import tpu as pltpu

N_DEV = 16
M = 4096
N = 1024
CH = M // N_DEV
HN = N // 2
SUB = 2
SR = CH // SUB
NH = N_DEV - 1

_POS = [0, 1, 8, 9, 15, 2, 7, 10, 14, 3, 6, 11, 13, 4, 5, 12]
_SUCC = [1, 5, 3, 7, 0, 9, 2, 11, 4, 13, 6, 15, 8, 14, 10, 12]
_PRED = [4, 0, 6, 2, 8, 1, 10, 3, 12, 5, 14, 7, 15, 9, 13, 11]


def kernel(x):
    def body(
        x_ref,
        out_ref,
        rs_send_f,
        rs_recv_f,
        rs_send_b,
        rs_recv_b,
        ag_f,
        ag_b,
        rs_ssem_f,
        rs_rsem_f,
        rs_ssem_b,
        rs_rsem_b,
        ag_ssem_f,
        ag_rsem_f,
        ag_ssem_b,
        ag_rsem_b,
    ):
        my = lax.axis_index("i")

        def lut(table):
            v = jnp.int32(table[0])
            for k in range(1, N_DEV):
                v = jnp.where(my == k, jnp.int32(table[k]), v)
            return v

        r = lut(_POS)
        right = lut(_SUCC)
        left = lut(_PRED)

        def xsub(c, s, fwd):
            cols = slice(0, HN) if fwd else slice(HN, N)
            return x_ref[pl.ds(c * CH + s * SR, SR), cols].astype(jnp.bfloat16)

        def rs_rdma(h, s, fwd):
            if fwd:
                return pltpu.make_async_remote_copy(
                    src_ref=rs_send_f.at[h, s],
                    dst_ref=rs_recv_f.at[h, s],
                    send_sem=rs_ssem_f.at[h * SUB + s],
                    recv_sem=rs_rsem_f.at[h * SUB + s],
                    device_id=(right,),
                    device_id_type=pl.DeviceIdType.MESH,
                )
            return pltpu.make_async_remote_copy(
                src_ref=rs_send_b.at[h, s],
                dst_ref=rs_recv_b.at[h, s],
                send_sem=rs_ssem_b.at[h * SUB + s],
                recv_sem=rs_rsem_b.at[h * SUB + s],
                device_id=(left,),
                device_id_type=pl.DeviceIdType.MESH,
            )

        def ag_rdma(h, s, fwd):
            if fwd:
                return pltpu.make_async_remote_copy(
                    src_ref=ag_f.at[NH if h == 0 else h - 1, s],
                    dst_ref=ag_f.at[h, s],
                    send_sem=ag_ssem_f.at[h * SUB + s],
                    recv_sem=ag_rsem_f.at[h * SUB + s],
                    device_id=(right,),
                    device_id_type=pl.DeviceIdType.MESH,
                )
            return pltpu.make_async_remote_copy(
                src_ref=ag_b.at[NH if h == 0 else h - 1, s],
                dst_ref=ag_b.at[h, s],
                send_sem=ag_ssem_b.at[h * SUB + s],
                recv_sem=ag_rsem_b.at[h * SUB + s],
                device_id=(left,),
                device_id_type=pl.DeviceIdType.MESH,
            )

        for s in range(SUB):
            rs_send_f[0, s, :, :] = xsub(r, s, True)
            rs_rdma(0, s, True).start()
            rs_send_b[0, s, :, :] = xsub(r, s, False)
            rs_rdma(0, s, False).start()

        for h in range(NH):
            cf = lax.rem(r - (h + 1) + 2 * N_DEV, N_DEV)
            cb = lax.rem(r + h + 1, N_DEV)
            for s in range(SUB):
                rs_rdma(h, s, True).wait_recv()
                if h < NH - 1:
                    rs_send_f[h + 1, s, :, :] = rs_recv_f[h, s] + xsub(cf, s, True)
                    rs_rdma(h + 1, s, True).start()
                else:
                    ag_f[NH, s, :, :] = rs_recv_f[h, s] + xsub(cf, s, True)
                    ag_rdma(0, s, True).start()
                rs_rdma(h, s, False).wait_recv()
                if h < NH - 1:
                    rs_send_b[h + 1, s, :, :] = rs_recv_b[h, s] + xsub(cb, s, False)
                    rs_rdma(h + 1, s, False).start()
                else:
                    ag_b[NH, s, :, :] = rs_recv_b[h, s] + xsub(cb, s, False)
                    ag_rdma(0, s, False).start()

        own_f = lax.rem(r + 1, N_DEV)
        own_b = lax.rem(r + N_DEV - 1, N_DEV)
        for s in range(SUB):
            out_ref[pl.ds(own_f * CH + s * SR, SR), 0:HN] = ag_f[NH, s].astype(
                jnp.float32
            )
            out_ref[pl.ds(own_b * CH + s * SR, SR), HN:N] = ag_b[NH, s].astype(
                jnp.float32
            )

        for h in range(NH):
            cf = lax.rem(r - h + 2 * N_DEV, N_DEV)
            cb = lax.rem(r + h, N_DEV)
            for s in range(SUB):
                ag_rdma(h, s, True).wait_recv()
                if h < NH - 1:
                    ag_rdma(h + 1, s, True).start()
                out_ref[pl.ds(cf * CH + s * SR, SR), 0:HN] = ag_f[h, s].astype(
                    jnp.float32
                )
                ag_rdma(h, s, False).wait_recv()
                if h < NH - 1:
                    ag_rdma(h + 1, s, False).start()
                out_ref[pl.ds(cb * CH + s * SR, SR), HN:N] = ag_b[h, s].astype(
                    jnp.float32
                )

        for h in range(NH):
            for s in range(SUB):
                rs_rdma(h, s, True).wait_send()
                rs_rdma(h, s, False).wait_send()
                ag_rdma(h, s, True).wait_send()
                ag_rdma(h, s, False).wait_send()

    rs_shape = (NH, SUB, SR, HN)
    ag_shape = (N_DEV, SUB, SR, HN)
    nsem = NH * SUB
    return pl.pallas_call(
        body,
        out_shape=jax.ShapeDtypeStruct((M, N), jnp.float32),
        in_specs=[pl.BlockSpec(memory_space=pltpu.VMEM)],
        out_specs=pl.BlockSpec(memory_space=pltpu.VMEM),
        scratch_shapes=[
            pltpu.VMEM(rs_shape, jnp.bfloat16),
            pltpu.VMEM(rs_shape, jnp.bfloat16),
            pltpu.VMEM(rs_shape, jnp.bfloat16),
            pltpu.VMEM(rs_shape, jnp.bfloat16),
            pltpu.VMEM(ag_shape, jnp.bfloat16),
            pltpu.VMEM(ag_shape, jnp.bfloat16),
            pltpu.SemaphoreType.DMA((nsem,)),
            pltpu.SemaphoreType.DMA((nsem,)),
            pltpu.SemaphoreType.DMA((nsem,)),
            pltpu.SemaphoreType.DMA((nsem,)),
            pltpu.SemaphoreType.DMA((nsem,)),
            pltpu.SemaphoreType.DMA((nsem,)),
            pltpu.SemaphoreType.DMA((nsem,)),
            pltpu.SemaphoreType.DMA((nsem,)),
        ],
        compiler_params=pltpu.CompilerParams(
            vmem_limit_bytes=100 * 1024 * 1024,
        ),
    )(x)


# device time: 124052 ns/iter; 2.0957x vs baseline; 1.0528x over previous
import jax
import jax.numpy as jnp
from jax import lax
from jax.experimental import pallas as pl
from jax.experimental.pallas import tpu as pltpu

N_DEV = 16
M = 4096
N = 1024
CH = M // N_DEV
HN = N // 2
SUB = 4
SR = CH // SUB
NH = N_DEV - 1

_POS = [0, 1, 8, 9, 15, 2, 7, 10, 14, 3, 6, 11, 13, 4, 5, 12]
_SUCC = [1, 5, 3, 7, 0, 9, 2, 11, 4, 13, 6, 15, 8, 14, 10, 12]
_PRED = [4, 0, 6, 2, 8, 1, 10, 3, 12, 5, 14, 7, 15, 9, 13, 11]


def kernel(x):
    def body(
        x_ref,
        out_ref,
        rs_send_f,
        rs_recv_f,
        rs_send_b,
        rs_recv_b,
        ag_f,
        ag_b,
        rs_ssem_f,
        rs_rsem_f,
        rs_ssem_b,
        rs_rsem_b,
        ag_ssem_f,
        ag_rsem_f,
        ag_ssem_b,
        ag_rsem_b,
    ):
        my = lax.axis_index("i")

        def lut(table):
            v = jnp.int32(table[0])
            for k in range(1, N_DEV):
                v = jnp.where(my == k, jnp.int32(table[k]), v)
            return v

        r = lut(_POS)
        right = lut(_SUCC)
        left = lut(_PRED)

        def xsub(c, s, fwd):
            cols = slice(0, HN) if fwd else slice(HN, N)
            return x_ref[pl.ds(c * CH + s * SR, SR), cols].astype(jnp.bfloat16)

        def rs_rdma(h, s, fwd):
            if fwd:
                return pltpu.make_async_remote_copy(
                    src_ref=rs_send_f.at[h, s],
                    dst_ref=rs_recv_f.at[h, s],
                    send_sem=rs_ssem_f.at[h * SUB + s],
                    recv_sem=rs_rsem_f.at[h * SUB + s],
                    device_id=(right,),
                    device_id_type=pl.DeviceIdType.MESH,
                )
            return pltpu.make_async_remote_copy(
                src_ref=rs_send_b.at[h, s],
                dst_ref=rs_recv_b.at[h, s],
                send_sem=rs_ssem_b.at[h * SUB + s],
                recv_sem=rs_rsem_b.at[h * SUB + s],
                device_id=(left,),
                device_id_type=pl.DeviceIdType.MESH,
            )

        def ag_rdma(h, s, fwd):
            if fwd:
                return pltpu.make_async_remote_copy(
                    src_ref=ag_f.at[NH if h == 0 else h - 1, s],
                    dst_ref=ag_f.at[h, s],
                    send_sem=ag_ssem_f.at[h * SUB + s],
                    recv_sem=ag_rsem_f.at[h * SUB + s],
                    device_id=(right,),
                    device_id_type=pl.DeviceIdType.MESH,
                )
            return pltpu.make_async_remote_copy(
                src_ref=ag_b.at[NH if h == 0 else h - 1, s],
                dst_ref=ag_b.at[h, s],
                send_sem=ag_ssem_b.at[h * SUB + s],
                recv_sem=ag_rsem_b.at[h * SUB + s],
                device_id=(left,),
                device_id_type=pl.DeviceIdType.MESH,
            )

        for s in range(SUB):
            rs_send_f[0, s, :, :] = xsub(r, s, True)
            rs_rdma(0, s, True).start()
            rs_send_b[0, s, :, :] = xsub(r, s, False)
            rs_rdma(0, s, False).start()

        for h in range(NH):
            cf = lax.rem(r - (h + 1) + 2 * N_DEV, N_DEV)
            cb = lax.rem(r + h + 1, N_DEV)
            for s in range(SUB):
                rs_rdma(h, s, True).wait_recv()
                if h < NH - 1:
                    rs_send_f[h + 1, s, :, :] = rs_recv_f[h, s] + xsub(cf, s, True)
                    rs_rdma(h + 1, s, True).start()
                else:
                    ag_f[NH, s, :, :] = rs_recv_f[h, s] + xsub(cf, s, True)
                    ag_rdma(0, s, True).start()
                rs_rdma(h, s, False).wait_recv()
                if h < NH - 1:
                    rs_send_b[h + 1, s, :, :] = rs_recv_b[h, s] + xsub(cb, s, False)
                    rs_rdma(h + 1, s, False).start()
                else:
                    ag_b[NH, s, :, :] = rs_recv_b[h, s] + xsub(cb, s, False)
                    ag_rdma(0, s, False).start()

        own_f = lax.rem(r + 1, N_DEV)
        own_b = lax.rem(r + N_DEV - 1, N_DEV)
        for s in range(SUB):
            out_ref[pl.ds(own_f * CH + s * SR, SR), 0:HN] = ag_f[NH, s].astype(
                jnp.float32
            )
            out_ref[pl.ds(own_b * CH + s * SR, SR), HN:N] = ag_b[NH, s].astype(
                jnp.float32
            )

        for h in range(NH):
            cf = lax.rem(r - h + 2 * N_DEV, N_DEV)
            cb = lax.rem(r + h, N_DEV)
            for s in range(SUB):
                ag_rdma(h, s, True).wait_recv()
                if h < NH - 1:
                    ag_rdma(h + 1, s, True).start()
                out_ref[pl.ds(cf * CH + s * SR, SR), 0:HN] = ag_f[h, s].astype(
                    jnp.float32
                )
                ag_rdma(h, s, False).wait_recv()
                if h < NH - 1:
                    ag_rdma(h + 1, s, False).start()
                out_ref[pl.ds(cb * CH + s * SR, SR), HN:N] = ag_b[h, s].astype(
                    jnp.float32
                )

        for h in range(NH):
            for s in range(SUB):
                rs_rdma(h, s, True).wait_send()
                rs_rdma(h, s, False).wait_send()
                ag_rdma(h, s, True).wait_send()
                ag_rdma(h, s, False).wait_send()

    rs_shape = (NH, SUB, SR, HN)
    ag_shape = (N_DEV, SUB, SR, HN)
    nsem = NH * SUB
    return pl.pallas_call(
        body,
        out_shape=jax.ShapeDtypeStruct((M, N), jnp.float32),
        in_specs=[pl.BlockSpec(memory_space=pltpu.VMEM)],
        out_specs=pl.BlockSpec(memory_space=pltpu.VMEM),
        scratch_shapes=[
            pltpu.VMEM(rs_shape, jnp.bfloat16),
            pltpu.VMEM(rs_shape, jnp.bfloat16),
            pltpu.VMEM(rs_shape, jnp.bfloat16),
            pltpu.VMEM(rs_shape, jnp.bfloat16),
            pltpu.VMEM(ag_shape, jnp.bfloat16),
            pltpu.VMEM(ag_shape, jnp.bfloat16),
            pltpu.SemaphoreType.DMA((nsem,)),
            pltpu.SemaphoreType.DMA((nsem,)),
            pltpu.SemaphoreType.DMA((nsem,)),
            pltpu.SemaphoreType.DMA((nsem,)),
            pltpu.SemaphoreType.DMA((nsem,)),
            pltpu.SemaphoreType.DMA((nsem,)),
            pltpu.SemaphoreType.DMA((nsem,)),
            pltpu.SemaphoreType.DMA((nsem,)),
        ],
        compiler_params=pltpu.CompilerParams(
            vmem_limit_bytes=100 * 1024 * 1024,
        ),
    )(x)


# device time: 123701 ns/iter; 2.1017x vs baseline; 1.0028x over previous
import jax
import jax.numpy as jnp
from jax import lax
from jax.experimental import pallas as pl
from jax.experimental.pallas import tpu as pltpu

N_DEV = 16
M = 4096
N = 1024
CH = M // N_DEV
HN = N // 2
SUB = 4
SR = CH // SUB
NH = N_DEV - 1

_POS = [0, 1, 8, 9, 15, 2, 7, 10, 14, 3, 6, 11, 13, 4, 5, 12]
_SUCC = [1, 5, 3, 7, 0, 9, 2, 11, 4, 13, 6, 15, 8, 14, 10, 12]
_PRED = [4, 0, 6, 2, 8, 1, 10, 3, 12, 5, 14, 7, 15, 9, 13, 11]


def kernel(x):
    def body(
        x_ref,
        out_ref,
        rs_send_f,
        rs_recv_f,
        rs_send_b,
        rs_recv_b,
        ag_f,
        ag_b,
        rs_ssem_f,
        rs_rsem_f,
        rs_ssem_b,
        rs_rsem_b,
        ag_ssem_f,
        ag_rsem_f,
        ag_ssem_b,
        ag_rsem_b,
    ):
        my = lax.axis_index("i")

        def lut(table):
            v = jnp.int32(table[0])
            for k in range(1, N_DEV):
                v = jnp.where(my == k, jnp.int32(table[k]), v)
            return v

        r = lut(_POS)
        right = lut(_SUCC)
        left = lut(_PRED)

        def xsub(c, s, fwd):
            cols = slice(0, HN) if fwd else slice(HN, N)
            return x_ref[pl.ds(c * CH + s * SR, SR), cols].astype(jnp.bfloat16)

        def rs_rdma(h, s, fwd):
            if fwd:
                return pltpu.make_async_remote_copy(
                    src_ref=rs_send_f.at[h, s],
                    dst_ref=rs_recv_f.at[h, s],
                    send_sem=rs_ssem_f.at[h * SUB + s],
                    recv_sem=rs_rsem_f.at[h * SUB + s],
                    device_id=(right,),
                    device_id_type=pl.DeviceIdType.MESH,
                )
            return pltpu.make_async_remote_copy(
                src_ref=rs_send_b.at[h, s],
                dst_ref=rs_recv_b.at[h, s],
                send_sem=rs_ssem_b.at[h * SUB + s],
                recv_sem=rs_rsem_b.at[h * SUB + s],
                device_id=(left,),
                device_id_type=pl.DeviceIdType.MESH,
            )

        def ag_rdma(h, s, fwd):
            if fwd:
                return pltpu.make_async_remote_copy(
                    src_ref=ag_f.at[NH if h == 0 else h - 1, s],
                    dst_ref=ag_f.at[h, s],
                    send_sem=ag_ssem_f.at[h * SUB + s],
                    recv_sem=ag_rsem_f.at[h * SUB + s],
                    device_id=(right,),
                    device_id_type=pl.DeviceIdType.MESH,
                )
            return pltpu.make_async_remote_copy(
                src_ref=ag_b.at[NH if h == 0 else h - 1, s],
                dst_ref=ag_b.at[h, s],
                send_sem=ag_ssem_b.at[h * SUB + s],
                recv_sem=ag_rsem_b.at[h * SUB + s],
                device_id=(left,),
                device_id_type=pl.DeviceIdType.MESH,
            )

        for s in range(SUB):
            rs_send_f[0, s, :, :] = xsub(r, s, True)
            rs_rdma(0, s, True).start()
            rs_send_b[0, s, :, :] = xsub(r, s, False)
            rs_rdma(0, s, False).start()

        for h in range(NH):
            cf = lax.rem(r - (h + 1) + 2 * N_DEV, N_DEV)
            cb = lax.rem(r + h + 1, N_DEV)
            for s in range(SUB):
                rs_rdma(h, s, True).wait_recv()
                if h < NH - 1:
                    rs_send_f[h + 1, s, :, :] = rs_recv_f[h, s] + xsub(cf, s, True)
                    rs_rdma(h + 1, s, True).start()
                else:
                    ag_f[NH, s, :, :] = rs_recv_f[h, s] + xsub(cf, s, True)
                    ag_rdma(0, s, True).start()
                rs_rdma(h, s, False).wait_recv()
                if h < NH - 1:
                    rs_send_b[h + 1, s, :, :] = rs_recv_b[h, s] + xsub(cb, s, False)
                    rs_rdma(h + 1, s, False).start()
                else:
                    ag_b[NH, s, :, :] = rs_recv_b[h, s] + xsub(cb, s, False)
                    ag_rdma(0, s, False).start()

        own_f = lax.rem(r + 1, N_DEV)
        own_b = lax.rem(r + N_DEV - 1, N_DEV)
        out_ref[pl.ds(own_f * CH, CH), 0:HN] = (
            ag_f[NH].astype(jnp.float32).reshape(CH, HN)
        )
        out_ref[pl.ds(own_b * CH, CH), HN:N] = (
            ag_b[NH].astype(jnp.float32).reshape(CH, HN)
        )

        for h in range(NH):
            cf = lax.rem(r - h + 2 * N_DEV, N_DEV)
            cb = lax.rem(r + h, N_DEV)
            for s in range(SUB):
                ag_rdma(h, s, True).wait_recv()
                if h < NH - 1:
                    ag_rdma(h + 1, s, True).start()
                ag_rdma(h, s, False).wait_recv()
                if h < NH - 1:
                    ag_rdma(h + 1, s, False).start()
            out_ref[pl.ds(cf * CH, CH), 0:HN] = (
                ag_f[h].astype(jnp.float32).reshape(CH, HN)
            )
            out_ref[pl.ds(cb * CH, CH), HN:N] = (
                ag_b[h].astype(jnp.float32).reshape(CH, HN)
            )

        for h in range(NH):
            for s in range(SUB):
                rs_rdma(h, s, True).wait_send()
                rs_rdma(h, s, False).wait_send()
                ag_rdma(h, s, True).wait_send()
                ag_rdma(h, s, False).wait_send()

    rs_shape = (NH, SUB, SR, HN)
    ag_shape = (N_DEV, SUB, SR, HN)
    nsem = NH * SUB
    return pl.pallas_call(
        body,
        out_shape=jax.ShapeDtypeStruct((M, N), jnp.float32),
        in_specs=[pl.BlockSpec(memory_space=pltpu.VMEM)],
        out_specs=pl.BlockSpec(memory_space=pltpu.VMEM),
        scratch_shapes=[
            pltpu.VMEM(rs_shape, jnp.bfloat16),
            pltpu.VMEM(rs_shape, jnp.bfloat16),
            pltpu.VMEM(rs_shape, jnp.bfloat16),
            pltpu.VMEM(rs_shape, jnp.bfloat16),
            pltpu.VMEM(ag_shape, jnp.bfloat16),
            pltpu.VMEM(ag_shape, jnp.bfloat16),
            pltpu.SemaphoreType.DMA((nsem,)),
            pltpu.SemaphoreType.DMA((nsem,)),
            pltpu.SemaphoreType.DMA((nsem,)),
            pltpu.SemaphoreType.DMA((nsem,)),
            pltpu.SemaphoreType.DMA((nsem,)),
            pltpu.SemaphoreType.DMA((nsem,)),
            pltpu.SemaphoreType.DMA((nsem,)),
            pltpu.SemaphoreType.DMA((nsem,)),
        ],
        compiler_params=pltpu.CompilerParams(
            vmem_limit_bytes=100 * 1024 * 1024,
        ),
    )(x)


# device time: 118472 ns/iter; 2.1944x vs baseline; 1.0441x over previous
import jax
import jax.numpy as jnp
from jax import lax
from jax.experimental import pallas as pl
from jax.experimental.pallas import tpu as pltpu

N_DEV = 16
M = 4096
N = 1024
CH = M // N_DEV
HN = N // 2
SUB = 4
SR = CH // SUB
NH = N_DEV - 1

_POS = [0, 1, 8, 9, 15, 2, 7, 10, 14, 3, 6, 11, 13, 4, 5, 12]
_SUCC = [1, 5, 3, 7, 0, 9, 2, 11, 4, 13, 6, 15, 8, 14, 10, 12]
_PRED = [4, 0, 6, 2, 8, 1, 10, 3, 12, 5, 14, 7, 15, 9, 13, 11]


def kernel(x):
    def body(
        x_ref,
        out_ref,
        rs_send_f,
        rs_recv_f,
        rs_send_b,
        rs_recv_b,
        ag_f,
        ag_b,
        rs_ssem_f,
        rs_rsem_f,
        rs_ssem_b,
        rs_rsem_b,
        ag_ssem_f,
        ag_rsem_f,
        ag_ssem_b,
        ag_rsem_b,
    ):
        my = lax.axis_index("i")

        def lut(table):
            v = jnp.int32(table[0])
            for k in range(1, N_DEV):
                v = jnp.where(my == k, jnp.int32(table[k]), v)
            return v

        r = lut(_POS)
        right = lut(_SUCC)
        left = lut(_PRED)

        def xsub(c, s, fwd):
            cols = slice(0, HN) if fwd else slice(HN, N)
            return x_ref[pl.ds(c * CH + s * SR, SR), cols].astype(jnp.bfloat16)

        def rs_rdma(h, s, fwd):
            if fwd:
                return pltpu.make_async_remote_copy(
                    src_ref=rs_send_f.at[h, s],
                    dst_ref=rs_recv_f.at[h, s],
                    send_sem=rs_ssem_f.at[h * SUB + s],
                    recv_sem=rs_rsem_f.at[h * SUB + s],
                    device_id=(right,),
                    device_id_type=pl.DeviceIdType.MESH,
                )
            return pltpu.make_async_remote_copy(
                src_ref=rs_send_b.at[h, s],
                dst_ref=rs_recv_b.at[h, s],
                send_sem=rs_ssem_b.at[h * SUB + s],
                recv_sem=rs_rsem_b.at[h * SUB + s],
                device_id=(left,),
                device_id_type=pl.DeviceIdType.MESH,
            )

        def ag_rdma(h, s, fwd):
            if fwd:
                return pltpu.make_async_remote_copy(
                    src_ref=ag_f.at[NH if h == 0 else h - 1, s],
                    dst_ref=ag_f.at[h, s],
                    send_sem=ag_ssem_f.at[h * SUB + s],
                    recv_sem=ag_rsem_f.at[h * SUB + s],
                    device_id=(right,),
                    device_id_type=pl.DeviceIdType.MESH,
                )
            return pltpu.make_async_remote_copy(
                src_ref=ag_b.at[NH if h == 0 else h - 1, s],
                dst_ref=ag_b.at[h, s],
                send_sem=ag_ssem_b.at[h * SUB + s],
                recv_sem=ag_rsem_b.at[h * SUB + s],
                device_id=(left,),
                device_id_type=pl.DeviceIdType.MESH,
            )

        barrier_sem = pltpu.get_barrier_semaphore()
        pl.semaphore_signal(
            barrier_sem, inc=1, device_id=(left,),
            device_id_type=pl.DeviceIdType.MESH,
        )
        pl.semaphore_signal(
            barrier_sem, inc=1, device_id=(right,),
            device_id_type=pl.DeviceIdType.MESH,
        )
        pl.semaphore_wait(barrier_sem, 2)

        for s in range(SUB):
            rs_send_f[0, s, :, :] = xsub(r, s, True)
            rs_rdma(0, s, True).start()
            rs_send_b[0, s, :, :] = xsub(r, s, False)
            rs_rdma(0, s, False).start()

        for h in range(NH):
            cf = lax.rem(r - (h + 1) + 2 * N_DEV, N_DEV)
            cb = lax.rem(r + h + 1, N_DEV)
            for s in range(SUB):
                rs_rdma(h, s, True).wait_recv()
                if h < NH - 1:
                    rs_send_f[h + 1, s, :, :] = rs_recv_f[h, s] + xsub(cf, s, True)
                    rs_rdma(h + 1, s, True).start()
                else:
                    ag_f[NH, s, :, :] = rs_recv_f[h, s] + xsub(cf, s, True)
                    ag_rdma(0, s, True).start()
                rs_rdma(h, s, False).wait_recv()
                if h < NH - 1:
                    rs_send_b[h + 1, s, :, :] = rs_recv_b[h, s] + xsub(cb, s, False)
                    rs_rdma(h + 1, s, False).start()
                else:
                    ag_b[NH, s, :, :] = rs_recv_b[h, s] + xsub(cb, s, False)
                    ag_rdma(0, s, False).start()

        own_f = lax.rem(r + 1, N_DEV)
        own_b = lax.rem(r + N_DEV - 1, N_DEV)
        out_ref[pl.ds(own_f * CH, CH), 0:HN] = (
            ag_f[NH].astype(jnp.float32).reshape(CH, HN)
        )
        out_ref[pl.ds(own_b * CH, CH), HN:N] = (
            ag_b[NH].astype(jnp.float32).reshape(CH, HN)
        )

        for h in range(NH):
            cf = lax.rem(r - h + 2 * N_DEV, N_DEV)
            cb = lax.rem(r + h, N_DEV)
            for s in range(SUB):
                ag_rdma(h, s, True).wait_recv()
                if h < NH - 1:
                    ag_rdma(h + 1, s, True).start()
                ag_rdma(h, s, False).wait_recv()
                if h < NH - 1:
                    ag_rdma(h + 1, s, False).start()
            out_ref[pl.ds(cf * CH, CH), 0:HN] = (
                ag_f[h].astype(jnp.float32).reshape(CH, HN)
            )
            out_ref[pl.ds(cb * CH, CH), HN:N] = (
                ag_b[h].astype(jnp.float32).reshape(CH, HN)
            )

        for h in range(NH):
            for s in range(SUB):
                rs_rdma(h, s, True).wait_send()
                rs_rdma(h, s, False).wait_send()
                ag_rdma(h, s, True).wait_send()
                ag_rdma(h, s, False).wait_send()

    rs_shape = (NH, SUB, SR, HN)
    ag_shape = (N_DEV, SUB, SR, HN)
    nsem = NH * SUB
    return pl.pallas_call(
        body,
        out_shape=jax.ShapeDtypeStruct((M, N), jnp.float32),
        in_specs=[pl.BlockSpec(memory_space=pltpu.VMEM)],
        out_specs=pl.BlockSpec(memory_space=pltpu.VMEM),
        scratch_shapes=[
            pltpu.VMEM(rs_shape, jnp.bfloat16),
            pltpu.VMEM(rs_shape, jnp.bfloat16),
            pltpu.VMEM(rs_shape, jnp.bfloat16),
            pltpu.VMEM(rs_shape, jnp.bfloat16),
            pltpu.VMEM(ag_shape, jnp.bfloat16),
            pltpu.VMEM(ag_shape, jnp.bfloat16),
            pltpu.SemaphoreType.DMA((nsem,)),
            pltpu.SemaphoreType.DMA((nsem,)),
            pltpu.SemaphoreType.DMA((nsem,)),
            pltpu.SemaphoreType.DMA((nsem,)),
            pltpu.SemaphoreType.DMA((nsem,)),
            pltpu.SemaphoreType.DMA((nsem,)),
            pltpu.SemaphoreType.DMA((nsem,)),
            pltpu.SemaphoreType.DMA((nsem,)),
        ],
        compiler_params=pltpu.CompilerParams(
            vmem_limit_bytes=100 * 1024 * 1024,
            collective_id=0,
        ),
    )(x)
